# Initial kernel scaffold; baseline (speedup 1.0000x reference)
import jax
import jax.numpy as jnp
from jax.experimental import pallas as pl


def kernel(x, atomic_subsystem_indices, W, b):
    raise NotImplementedError("write your pallas kernel here")


# speedup vs baseline: 1.1172x; 1.1172x over previous
import functools

import jax
import jax.numpy as jnp
from jax import lax
from jax.experimental import pallas as pl
from jax.experimental.pallas import tpu as pltpu
from jax.experimental.pallas import tpu_sc as plsc

_N_ATOMS = 320000
_N_BASIS = 128
_N_MOL = 10000

_BLK = 6400


def _mv_body(x_ref, w_ref, b_ref, y_ref):
    y_ref[...] = (
        jnp.sum(x_ref[...] * w_ref[...], axis=1, keepdims=True) + b_ref[...]
    )


def _matvec(x, W, b2d):
    n = x.shape[0]
    return pl.pallas_call(
        _mv_body,
        grid=(n // _BLK,),
        in_specs=[
            pl.BlockSpec((_BLK, _N_BASIS), lambda i: (i, 0)),
            pl.BlockSpec((1, _N_BASIS), lambda i: (0, 0)),
            pl.BlockSpec((1, 1), lambda i: (0, 0)),
        ],
        out_specs=pl.BlockSpec((_BLK, 1), lambda i: (i, 0)),
        out_shape=jax.ShapeDtypeStruct((n, 1), jnp.float32),
    )(x, W, b2d)


_NSUB = 16
_CHUNK = _N_ATOMS // _NSUB
_ROWS = (_CHUNK + 127) // 128
_CPAD = _ROWS * 128
_HALF = _N_MOL // 2
_ACC = 5120
_SLC = _ACC // _NSUB


def _seg_body(y_hbm, idx_hbm, out_hbm, y_v, idx_v, vals_v, lidx_v, z_v, acc_sh):
    c = lax.axis_index("c")
    s = lax.axis_index("s")
    base = s * _CHUNK

    zf = jnp.zeros((16,), jnp.float32)
    zi = jnp.zeros((16,), jnp.int32)
    for i in range(_SLC // 16):
        z_v[pl.ds(i * 16, 16)] = zf
    pltpu.sync_copy(z_v, acc_sh.at[pl.ds(s * _SLC, _SLC)])
    for i in range((_CPAD - _CHUNK) // 16):
        y_v[pl.ds(_CHUNK + i * 16, 16)] = zf
        idx_v[pl.ds(_CHUNK + i * 16, 16)] = zi

    pltpu.sync_copy(y_hbm.at[pl.ds(base, _CHUNK)], y_v.at[pl.ds(0, _CHUNK)])
    pltpu.sync_copy(idx_hbm.at[pl.ds(base, _CHUNK)], idx_v.at[pl.ds(0, _CHUNK)])

    lo = c * _HALF

    def row_body(j, carry):
        for k2 in range(8):
            off = j * 128 + k2 * 16
            yv = y_v[pl.ds(off, 16)]
            iv = idx_v[pl.ds(off, 16)]
            li = iv - lo
            inr = (li >= 0) & (li < _HALF)
            li = jnp.clip(li, 0, _ACC - 1)
            vals_v[j, pl.ds(k2 * 16, 16)] = jnp.where(inr, yv, 0.0)
            lidx_v[j, pl.ds(k2 * 16, 16)] = li
        return carry

    lax.fori_loop(0, _ROWS, row_body, 0)

    plsc.subcore_barrier()

    def sc_body(j, carry):
        pltpu.sync_copy(vals_v.at[j], acc_sh.at[lidx_v.at[j]], add=True)
        return carry

    lax.fori_loop(0, _ROWS, sc_body, 0)

    plsc.subcore_barrier()

    pltpu.sync_copy(acc_sh.at[pl.ds(s * _SLC, _SLC)], z_v)
    pltpu.sync_copy(z_v, out_hbm.at[pl.ds(c * _ACC + s * _SLC, _SLC)])


def _segsum(y, idx):
    k = pl.kernel(
        _seg_body,
        mesh=plsc.VectorSubcoreMesh(core_axis_name="c", subcore_axis_name="s"),
        out_type=jax.ShapeDtypeStruct((2 * _ACC,), jnp.float32),
        scratch_types=[
            pltpu.VMEM((_CPAD,), jnp.float32),
            pltpu.VMEM((_CPAD,), jnp.int32),
            pltpu.VMEM((_ROWS, 128), jnp.float32),
            pltpu.VMEM((_ROWS, 128), jnp.int32),
            pltpu.VMEM((_SLC,), jnp.float32),
            pltpu.VMEM_SHARED((_ACC,), jnp.float32),
        ],
    )
    return k(y, idx)


def kernel(x, atomic_subsystem_indices, W, b):
    idx = atomic_subsystem_indices.astype(jnp.int32)
    y = _matvec(x, W, b.reshape(1, 1))
    part = _segsum(y.reshape(-1), idx)
    return part.reshape(2, _ACC)[:, :_HALF].reshape(_N_MOL, 1)
